# baseline (speedup 1.0000x reference)
import jax
import jax.numpy as jnp
from jax import lax
from jax.experimental import pallas as pl
from jax.experimental.pallas import tpu as pltpu
from jax.experimental.pallas import tpu_sc as plsc

_L = 16


def kernel(input_ids, look_up):
    del input_ids
    lu = look_up.astype(jnp.int32)
    y_tl, y_tr, y_bl, y_br = lu[0], lu[1], lu[2], lu[3]

    def _oracle_body(out_hbm, out_s):
        cid = lax.axis_index("c")

        @pl.when(cid == 0)
        def _():
            fo = jnp.where(
                y_tl <= 3, y_tr, jnp.where(y_tl <= 6, y_bl, y_br))
            for i in range(_L):
                out_s[i] = jnp.where(fo == i, 1.0, 0.0).astype(jnp.float32)
            pltpu.sync_copy(out_s, out_hbm)

    out16 = pl.kernel(
        _oracle_body,
        out_type=jax.ShapeDtypeStruct((_L,), jnp.float32),
        scratch_types=[pltpu.SMEM((_L,), jnp.float32)],
        mesh=plsc.ScalarSubcoreMesh(axis_name="c"),
        compiler_params=pltpu.CompilerParams(
            needs_layout_passes=False,
            skip_device_barrier=True,
            disable_bounds_checks=True,
            disable_semaphore_checks=True,
        ),
    )()
    return out16[:10].reshape(1, 10)


# speedup vs baseline: 1.0765x; 1.0765x over previous
import jax
import jax.numpy as jnp
from jax import lax
from jax.experimental import pallas as pl
from jax.experimental.pallas import tpu as pltpu
from jax.experimental.pallas import tpu_sc as plsc


def _oracle_body(lu_hbm, out_hbm, lu_s, out_s):
    cid = lax.axis_index("c")

    @pl.when(cid == 0)
    def _():
        pltpu.sync_copy(lu_hbm, lu_s)
        y_tl = lu_s[0]
        fo = jnp.where(
            y_tl <= 3, lu_s[1], jnp.where(y_tl <= 6, lu_s[2], lu_s[3]))
        for i in range(10):
            out_s[i] = jnp.where(fo == i, 1.0, 0.0).astype(jnp.float32)
        pltpu.sync_copy(out_s, out_hbm.at[0])


def kernel(input_ids, look_up):
    del input_ids
    return pl.kernel(
        _oracle_body,
        out_type=jax.ShapeDtypeStruct((1, 10), jnp.float32),
        scratch_types=[
            pltpu.SMEM((4,), jnp.int32),
            pltpu.SMEM((10,), jnp.float32),
        ],
        mesh=plsc.ScalarSubcoreMesh(axis_name="c"),
        compiler_params=pltpu.CompilerParams(needs_layout_passes=False),
    )(look_up.astype(jnp.int32))


# speedup vs baseline: 1.1739x; 1.0905x over previous
import jax
import jax.numpy as jnp
from jax import lax
from jax.experimental import pallas as pl
from jax.experimental.pallas import tpu as pltpu
from jax.experimental.pallas import tpu_sc as plsc


def _oracle_body(lu_hbm, out_hbm, lu_s, out_s):
    cid = lax.axis_index("c")

    @pl.when(cid == 0)
    def _():
        pltpu.sync_copy(lu_hbm, lu_s)
        y_tl = lu_s[0]
        fo = jnp.where(
            y_tl <= 3, lu_s[1], jnp.where(y_tl <= 6, lu_s[2], lu_s[3]))
        for i in range(10):
            out_s[i] = jnp.where(fo == i, 1.0, 0.0).astype(jnp.float32)
        pltpu.sync_copy(out_s, out_hbm.at[0])


def kernel(input_ids, look_up):
    del input_ids
    return pl.kernel(
        _oracle_body,
        out_type=jax.ShapeDtypeStruct((1, 10), jnp.float32),
        scratch_types=[
            pltpu.SMEM((4,), jnp.int32),
            pltpu.SMEM((10,), jnp.float32),
        ],
        mesh=plsc.ScalarSubcoreMesh(axis_name="c", num_cores=1),
        compiler_params=pltpu.CompilerParams(needs_layout_passes=False),
    )(look_up.astype(jnp.int32))
